# baseline (speedup 1.0000x reference)
import functools

import jax
import jax.numpy as jnp
import numpy as np
from jax import lax
from jax.experimental import pallas as pl
from jax.experimental.pallas import tpu as pltpu
from jax.experimental.pallas import tpu_sc as plsc

N_NODES = 10000
N_EDGES = 320000
D = 128

NC = 2
NS = 16
NW = NC * NS

WIN = 64
G = 16
GW = G * WIN
N_WINDOWS = 5120
WPW = N_WINDOWS // NW
NG = WPW // G
IDX_WINDOWS = N_WINDOWS + G
E_PAD = IDX_WINDOWS * WIN
N_PAD = 10240
RPT = N_PAD // NS
NBUF = 4
LAG = 2


def _sc_agg_body(compute_cnt, x_hbm, src_hbm, dst_hbm, *refs):
    if compute_cnt:
        (agg_out, cnt_out, sidx0, sidx1, didx0, didx1, d2d0, d2d1,
         rows0, rows1, rows2, rows3, pidx, pidx_lo, ones_v,
         agg_sh, cnt_sh,
         gsem0, gsem1, gsem2, gsem3, ssem0, ssem1, ssem2, ssem3,
         csem0, csem1, csem2, csem3, isem0, isem1) = refs
        csem = (csem0, csem1, csem2, csem3)
    else:
        (agg_out, sidx0, sidx1, didx0, didx1, d2d0, d2d1,
         rows0, rows1, rows2, rows3, pidx, pidx_lo,
         agg_sh,
         gsem0, gsem1, gsem2, gsem3, ssem0, ssem1, ssem2, ssem3,
         isem0, isem1) = refs

    sidx = (sidx0, sidx1)
    didx = (didx0, didx1)
    d2d = (d2d0, d2d1)
    rows = (rows0, rows1, rows2, rows3)
    gsem = (gsem0, gsem1, gsem2, gsem3)
    ssem = (ssem0, ssem1, ssem2, ssem3)
    isem = (isem0, isem1)

    c = lax.axis_index("c")
    s = lax.axis_index("s")
    wid = s * NC + c
    ebase = wid * WPW * WIN

    z16 = jnp.zeros((16,), jnp.float32)
    iota16 = lax.iota(jnp.int32, 16)
    for j in range(WIN // 16):
        pidx[pl.ds(j * 16, 16)] = iota16 + (
            N_NODES + (s * WIN + j * 16) % (N_PAD - N_NODES - 16))
        pidx_lo[pl.ds(j * 16, 16)] = iota16 + s * WIN + j * 16

    def _zrow(i, carry):
        for j in range(D // 16):
            rows0[i, pl.ds(j * 16, 16)] = z16
        return carry

    lax.fori_loop(0, WIN, _zrow, 0)

    if compute_cnt:
        one16 = jnp.ones((16,), jnp.float32)
        for j in range(WIN // 16):
            ones_v[pl.ds(j * 16, 16)] = one16

    pltpu.async_copy(src_hbm.at[pl.ds(ebase, GW)], sidx0, isem0)
    pltpu.async_copy(dst_hbm.at[pl.ds(ebase, GW)], didx0, isem0)
    pltpu.async_copy(x_hbm.at[pidx_lo], rows2, gsem2)
    pltpu.async_copy(x_hbm.at[pidx_lo], rows3, gsem3)

    base = s * RPT
    for t in range(RPT // WIN):
        pltpu.sync_copy(rows0, agg_sh.at[pl.ds(base + t * WIN, WIN)])
    if compute_cnt:
        for t in range(RPT // D):
            pltpu.sync_copy(rows0.at[0], cnt_sh.at[pl.ds(base + t * D, D)])
    plsc.subcore_barrier()

    def wait_gather(b):
        pltpu.make_async_copy(x_hbm.at[pidx_lo], rows[b], gsem[b]).wait()

    def wait_scatter(b):
        pltpu.make_async_copy(rows[b], agg_sh.at[pidx], ssem[b]).wait()
        if compute_cnt:
            pltpu.make_async_copy(ones_v, cnt_sh.at[pidx], csem[b]).wait()

    def wait_idx(gb, eoff):
        pltpu.make_async_copy(src_hbm.at[pl.ds(eoff, GW)], sidx[gb],
                              isem[gb]).wait()
        pltpu.make_async_copy(dst_hbm.at[pl.ds(eoff, GW)], didx[gb],
                              isem[gb]).wait()

    def fill_d2d(gb):
        for k in range(G):
            for j in range(WIN // 16):
                d2d[gb][k, pl.ds(j * 16, 16)] = (
                    didx[gb][pl.ds(k * WIN + j * 16, 16)])

    for j in range(WIN // 16):
        d2d1[G - 2, pl.ds(j * 16, 16)] = pidx[pl.ds(j * 16, 16)]
        d2d1[G - 1, pl.ds(j * 16, 16)] = pidx[pl.ds(j * 16, 16)]
    for b in range(LAG):
        pltpu.async_copy(rows[b], agg_sh.at[pidx], ssem[b], add=True)
        if compute_cnt:
            pltpu.async_copy(ones_v, cnt_sh.at[pidx], csem[b], add=True)

    def pair_body(gg, carry):
        for gpar in range(2):
            gb = gpar
            g = 2 * gg + gpar
            geoff = ebase + g * GW
            wait_idx(gb, geoff)
            fill_d2d(gb)
            for k in range(G):
                b = k % NBUF
                bv = (k + LAG) % NBUF
                wait_scatter(b)
                pltpu.async_copy(
                    x_hbm.at[sidx[gb].at[pl.ds(k * WIN, WIN)]],
                    rows[b], gsem[b])
                wait_gather(bv)
                if k < LAG:
                    dv = d2d[gb ^ 1].at[G - LAG + k]
                else:
                    dv = d2d[gb].at[k - LAG]
                pltpu.async_copy(rows[bv], agg_sh.at[dv], ssem[bv], add=True)
                if compute_cnt:
                    pltpu.async_copy(ones_v, cnt_sh.at[dv], csem[bv],
                                     add=True)
                if k == 3:
                    nxt = ebase + (g + 1) * GW
                    pltpu.async_copy(src_hbm.at[pl.ds(nxt, GW)], sidx[gb ^ 1],
                                     isem[gb ^ 1])
                    pltpu.async_copy(dst_hbm.at[pl.ds(nxt, GW)], didx[gb ^ 1],
                                     isem[gb ^ 1])
        return carry

    lax.fori_loop(0, NG // 2, pair_body, 0)

    for e in range(LAG):
        kv = G - LAG + e
        bv = (WPW - LAG + e) % NBUF
        wait_gather(bv)
        pltpu.async_copy(rows[bv], agg_sh.at[d2d1.at[kv]], ssem[bv],
                         add=True)
        if compute_cnt:
            pltpu.async_copy(ones_v, cnt_sh.at[d2d1.at[kv]], csem[bv],
                             add=True)
    for b in range(NBUF):
        wait_scatter(b)
    wait_idx(0, ebase + NG * GW)

    plsc.subcore_barrier()

    for t in range(RPT // 128):
        pltpu.sync_copy(agg_sh.at[pl.ds(base + t * 128, 128)],
                        agg_out.at[c, pl.ds(base + t * 128, 128)])
    if compute_cnt:
        pltpu.sync_copy(cnt_sh.at[pl.ds(base, RPT)],
                        cnt_out.at[c, pl.ds(base, RPT)])


def _make_sc_agg(compute_cnt):
    mesh = plsc.VectorSubcoreMesh(core_axis_name="c", subcore_axis_name="s",
                                  num_cores=NC, num_subcores=NS)
    out_type = [jax.ShapeDtypeStruct((NC, N_PAD, D), jnp.float32)]
    if compute_cnt:
        out_type.append(jax.ShapeDtypeStruct((NC, N_PAD), jnp.float32))
    scratch = [
        pltpu.VMEM((GW,), jnp.int32),
        pltpu.VMEM((GW,), jnp.int32),
        pltpu.VMEM((GW,), jnp.int32),
        pltpu.VMEM((GW,), jnp.int32),
        pltpu.VMEM((G, WIN), jnp.int32),
        pltpu.VMEM((G, WIN), jnp.int32),
        pltpu.VMEM((WIN, D), jnp.float32),
        pltpu.VMEM((WIN, D), jnp.float32),
        pltpu.VMEM((WIN, D), jnp.float32),
        pltpu.VMEM((WIN, D), jnp.float32),
        pltpu.VMEM((WIN,), jnp.int32),
        pltpu.VMEM((WIN,), jnp.int32),
    ]
    if compute_cnt:
        scratch.append(pltpu.VMEM((WIN,), jnp.float32))
    scratch.append(pltpu.VMEM_SHARED((N_PAD, D), jnp.float32))
    if compute_cnt:
        scratch.append(pltpu.VMEM_SHARED((N_PAD,), jnp.float32))
    nsem = 10 + (4 if compute_cnt else 0)
    scratch += [pltpu.SemaphoreType.DMA] * nsem

    return pl.kernel(
        functools.partial(_sc_agg_body, compute_cnt),
        out_type=tuple(out_type),
        mesh=mesh,
        scratch_types=scratch,
        name=f"sage_sc_agg_cnt{int(compute_cnt)}",
    )


_SC_AGG_CNT = _make_sc_agg(True)
_SC_AGG = _make_sc_agg(False)


def _combine_body(relu, a, cc, xr, wn, ws, br, o):
    i = pl.program_id(0)
    cnt = (cc[0:1, pl.ds(i * _R_BLK, _R_BLK)]
           + cc[1:2, pl.ds(i * _R_BLK, _R_BLK)])
    r = 1.0 / jnp.maximum(cnt, 1.0)
    r_col = jnp.transpose(r, (1, 0))
    mean = (a[0] + a[1]) * r_col
    acc = jnp.dot(mean, wn[...], preferred_element_type=jnp.float32)
    acc = acc + jnp.dot(xr[...], ws[...], preferred_element_type=jnp.float32)
    acc = acc + br[...]
    if relu:
        acc = jnp.maximum(acc, 0.0)
    o[...] = acc


_R_BLK = 2048


def _combine(agg, cnt2, xr, wn, ws, br, relu):
    grid = (N_PAD // _R_BLK,)
    row_spec = pl.BlockSpec((_R_BLK, D), lambda i: (i, 0))
    a_spec = pl.BlockSpec((NC, _R_BLK, D), lambda i: (0, i, 0))
    c_spec = pl.BlockSpec((NC, N_PAD), lambda i: (0, 0))
    w_spec = pl.BlockSpec((D, D), lambda i: (0, 0))
    b_spec = pl.BlockSpec((1, D), lambda i: (0, 0))
    return pl.pallas_call(
        functools.partial(_combine_body, relu),
        grid=grid,
        in_specs=[a_spec, c_spec, row_spec, w_spec, w_spec, b_spec],
        out_specs=row_spec,
        out_shape=jax.ShapeDtypeStruct((N_NODES, D), jnp.float32),
        name=f"sage_combine_relu{int(relu)}",
    )(agg, cnt2, xr, wn, ws, br)


_PAD_N = E_PAD - N_EDGES
_PAD_SRC = jnp.asarray(np.arange(_PAD_N, dtype=np.int32) % N_NODES)
_PAD_DST = jnp.asarray(
    N_NODES + np.arange(_PAD_N, dtype=np.int32) % (N_PAD - N_NODES))


def kernel(x, edge_index, W1_self, W1_neigh, b1, W2_self, W2_neigh, b2):
    flat = lax.optimization_barrier(jnp.reshape(edge_index, (2 * N_EDGES,)))
    src_p = jnp.concatenate([flat[:N_EDGES], _PAD_SRC])
    dst_p = jnp.concatenate([flat[N_EDGES:], _PAD_DST])

    agg1, cnt = _SC_AGG_CNT(x, src_p, dst_p)
    b1r = b1.reshape(1, D)
    b2r = b2.reshape(1, D)

    h = _combine(agg1, cnt, x, W1_neigh, W1_self, b1r, relu=True)
    (agg2,) = _SC_AGG(h, src_p, dst_p)
    out = _combine(agg2, cnt, h, W2_neigh, W2_self, b2r, relu=False)
    return out


# speedup vs baseline: 1.0058x; 1.0058x over previous
import functools

import jax
import jax.numpy as jnp
import numpy as np
from jax import lax
from jax.experimental import pallas as pl
from jax.experimental.pallas import tpu as pltpu
from jax.experimental.pallas import tpu_sc as plsc

N_NODES = 10000
N_EDGES = 320000
D = 128

NC = 2
NS = 16
NW = NC * NS

WIN = 64
G = 8
GW = G * WIN
N_WINDOWS = 5120
WPW = N_WINDOWS // NW
NG = WPW // G
IDX_WINDOWS = N_WINDOWS + G
E_PAD = IDX_WINDOWS * WIN
N_PAD = 10240
RPT = N_PAD // NS
NBUF = 4
LAG = 2


def _sc_agg_body(compute_cnt, x_hbm, src_hbm, dst_hbm, *refs):
    if compute_cnt:
        (agg_out, cnt_out, sidx0, sidx1, didx0, didx1, d2d0, d2d1,
         rows0, rows1, rows2, rows3, pidx, pidx_lo, ones_v,
         agg_sh, cnt_sh,
         gsem0, gsem1, gsem2, gsem3, ssem0, ssem1, ssem2, ssem3,
         csem0, csem1, csem2, csem3, isem0, isem1) = refs
        csem = (csem0, csem1, csem2, csem3)
    else:
        (agg_out, sidx0, sidx1, didx0, didx1, d2d0, d2d1,
         rows0, rows1, rows2, rows3, pidx, pidx_lo,
         agg_sh,
         gsem0, gsem1, gsem2, gsem3, ssem0, ssem1, ssem2, ssem3,
         isem0, isem1) = refs

    sidx = (sidx0, sidx1)
    didx = (didx0, didx1)
    d2d = (d2d0, d2d1)
    rows = (rows0, rows1, rows2, rows3)
    gsem = (gsem0, gsem1, gsem2, gsem3)
    ssem = (ssem0, ssem1, ssem2, ssem3)
    isem = (isem0, isem1)

    c = lax.axis_index("c")
    s = lax.axis_index("s")
    wid = s * NC + c
    ebase = wid * WPW * WIN

    z16 = jnp.zeros((16,), jnp.float32)
    iota16 = lax.iota(jnp.int32, 16)
    for j in range(WIN // 16):
        pidx[pl.ds(j * 16, 16)] = iota16 + (
            N_NODES + (s * WIN + j * 16) % (N_PAD - N_NODES - 16))
        pidx_lo[pl.ds(j * 16, 16)] = iota16 + s * WIN + j * 16

    def _zrow(i, carry):
        for j in range(D // 16):
            rows0[i, pl.ds(j * 16, 16)] = z16
        return carry

    lax.fori_loop(0, WIN, _zrow, 0)

    if compute_cnt:
        one16 = jnp.ones((16,), jnp.float32)
        for j in range(WIN // 16):
            ones_v[pl.ds(j * 16, 16)] = one16

    pltpu.async_copy(src_hbm.at[pl.ds(ebase, GW)], sidx0, isem0)
    pltpu.async_copy(dst_hbm.at[pl.ds(ebase, GW)], didx0, isem0)
    pltpu.async_copy(x_hbm.at[pidx_lo], rows2, gsem2)
    pltpu.async_copy(x_hbm.at[pidx_lo], rows3, gsem3)

    base = s * RPT
    for t in range(RPT // WIN):
        pltpu.sync_copy(rows0, agg_sh.at[pl.ds(base + t * WIN, WIN)])
    if compute_cnt:
        for t in range(RPT // D):
            pltpu.sync_copy(rows0.at[0], cnt_sh.at[pl.ds(base + t * D, D)])
    plsc.subcore_barrier()

    def wait_gather(b):
        pltpu.make_async_copy(x_hbm.at[pidx_lo], rows[b], gsem[b]).wait()

    def wait_scatter(b):
        pltpu.make_async_copy(rows[b], agg_sh.at[pidx], ssem[b]).wait()
        if compute_cnt:
            pltpu.make_async_copy(ones_v, cnt_sh.at[pidx], csem[b]).wait()

    def wait_idx(gb, eoff):
        pltpu.make_async_copy(src_hbm.at[pl.ds(eoff, GW)], sidx[gb],
                              isem[gb]).wait()
        pltpu.make_async_copy(dst_hbm.at[pl.ds(eoff, GW)], didx[gb],
                              isem[gb]).wait()

    def fill_d2d(gb):
        for k in range(G):
            for j in range(WIN // 16):
                d2d[gb][k, pl.ds(j * 16, 16)] = (
                    didx[gb][pl.ds(k * WIN + j * 16, 16)])

    for j in range(WIN // 16):
        d2d1[G - 2, pl.ds(j * 16, 16)] = pidx[pl.ds(j * 16, 16)]
        d2d1[G - 1, pl.ds(j * 16, 16)] = pidx[pl.ds(j * 16, 16)]
    for b in range(LAG):
        pltpu.async_copy(rows[b], agg_sh.at[pidx], ssem[b], add=True)
        if compute_cnt:
            pltpu.async_copy(ones_v, cnt_sh.at[pidx], csem[b], add=True)

    def pair_body(gg, carry):
        for gpar in range(2):
            gb = gpar
            g = 2 * gg + gpar
            geoff = ebase + g * GW
            wait_idx(gb, geoff)
            fill_d2d(gb)
            for k in range(G):
                b = k % NBUF
                bv = (k + LAG) % NBUF
                wait_scatter(b)
                pltpu.async_copy(
                    x_hbm.at[sidx[gb].at[pl.ds(k * WIN, WIN)]],
                    rows[b], gsem[b])
                wait_gather(bv)
                if k < LAG:
                    dv = d2d[gb ^ 1].at[G - LAG + k]
                else:
                    dv = d2d[gb].at[k - LAG]
                pltpu.async_copy(rows[bv], agg_sh.at[dv], ssem[bv], add=True)
                if compute_cnt:
                    pltpu.async_copy(ones_v, cnt_sh.at[dv], csem[bv],
                                     add=True)
                if k == 3:
                    nxt = ebase + (g + 1) * GW
                    pltpu.async_copy(src_hbm.at[pl.ds(nxt, GW)], sidx[gb ^ 1],
                                     isem[gb ^ 1])
                    pltpu.async_copy(dst_hbm.at[pl.ds(nxt, GW)], didx[gb ^ 1],
                                     isem[gb ^ 1])
        return carry

    lax.fori_loop(0, NG // 2, pair_body, 0)

    for e in range(LAG):
        kv = G - LAG + e
        bv = (WPW - LAG + e) % NBUF
        wait_gather(bv)
        pltpu.async_copy(rows[bv], agg_sh.at[d2d1.at[kv]], ssem[bv],
                         add=True)
        if compute_cnt:
            pltpu.async_copy(ones_v, cnt_sh.at[d2d1.at[kv]], csem[bv],
                             add=True)
    for b in range(NBUF):
        wait_scatter(b)
    wait_idx(0, ebase + NG * GW)

    plsc.subcore_barrier()

    for t in range(RPT // 128):
        pltpu.sync_copy(agg_sh.at[pl.ds(base + t * 128, 128)],
                        agg_out.at[c, pl.ds(base + t * 128, 128)])
    if compute_cnt:
        pltpu.sync_copy(cnt_sh.at[pl.ds(base, RPT)],
                        cnt_out.at[c, pl.ds(base, RPT)])


def _make_sc_agg(compute_cnt):
    mesh = plsc.VectorSubcoreMesh(core_axis_name="c", subcore_axis_name="s",
                                  num_cores=NC, num_subcores=NS)
    out_type = [jax.ShapeDtypeStruct((NC, N_PAD, D), jnp.float32)]
    if compute_cnt:
        out_type.append(jax.ShapeDtypeStruct((NC, N_PAD), jnp.float32))
    scratch = [
        pltpu.VMEM((GW,), jnp.int32),
        pltpu.VMEM((GW,), jnp.int32),
        pltpu.VMEM((GW,), jnp.int32),
        pltpu.VMEM((GW,), jnp.int32),
        pltpu.VMEM((G, WIN), jnp.int32),
        pltpu.VMEM((G, WIN), jnp.int32),
        pltpu.VMEM((WIN, D), jnp.float32),
        pltpu.VMEM((WIN, D), jnp.float32),
        pltpu.VMEM((WIN, D), jnp.float32),
        pltpu.VMEM((WIN, D), jnp.float32),
        pltpu.VMEM((WIN,), jnp.int32),
        pltpu.VMEM((WIN,), jnp.int32),
    ]
    if compute_cnt:
        scratch.append(pltpu.VMEM((WIN,), jnp.float32))
    scratch.append(pltpu.VMEM_SHARED((N_PAD, D), jnp.float32))
    if compute_cnt:
        scratch.append(pltpu.VMEM_SHARED((N_PAD,), jnp.float32))
    nsem = 10 + (4 if compute_cnt else 0)
    scratch += [pltpu.SemaphoreType.DMA] * nsem

    return pl.kernel(
        functools.partial(_sc_agg_body, compute_cnt),
        out_type=tuple(out_type),
        mesh=mesh,
        scratch_types=scratch,
        name=f"sage_sc_agg_cnt{int(compute_cnt)}",
    )


_SC_AGG_CNT = _make_sc_agg(True)
_SC_AGG = _make_sc_agg(False)


def _combine_body(relu, a, cc, xr, wn, ws, br, o):
    i = pl.program_id(0)
    cnt = (cc[0:1, pl.ds(i * _R_BLK, _R_BLK)]
           + cc[1:2, pl.ds(i * _R_BLK, _R_BLK)])
    r = 1.0 / jnp.maximum(cnt, 1.0)
    r_col = jnp.transpose(r, (1, 0))
    mean = (a[0] + a[1]) * r_col
    acc = jnp.dot(mean, wn[...], preferred_element_type=jnp.float32)
    acc = acc + jnp.dot(xr[...], ws[...], preferred_element_type=jnp.float32)
    acc = acc + br[...]
    if relu:
        acc = jnp.maximum(acc, 0.0)
    o[...] = acc


_R_BLK = 2048


def _combine(agg, cnt2, xr, wn, ws, br, relu):
    grid = (N_PAD // _R_BLK,)
    row_spec = pl.BlockSpec((_R_BLK, D), lambda i: (i, 0))
    a_spec = pl.BlockSpec((NC, _R_BLK, D), lambda i: (0, i, 0))
    c_spec = pl.BlockSpec((NC, N_PAD), lambda i: (0, 0))
    w_spec = pl.BlockSpec((D, D), lambda i: (0, 0))
    b_spec = pl.BlockSpec((1, D), lambda i: (0, 0))
    return pl.pallas_call(
        functools.partial(_combine_body, relu),
        grid=grid,
        in_specs=[a_spec, c_spec, row_spec, w_spec, w_spec, b_spec],
        out_specs=row_spec,
        out_shape=jax.ShapeDtypeStruct((N_NODES, D), jnp.float32),
        name=f"sage_combine_relu{int(relu)}",
    )(agg, cnt2, xr, wn, ws, br)


_PAD_N = E_PAD - N_EDGES
_PAD_SRC = jnp.asarray(np.arange(_PAD_N, dtype=np.int32) % N_NODES)
_PAD_DST = jnp.asarray(
    N_NODES + np.arange(_PAD_N, dtype=np.int32) % (N_PAD - N_NODES))


def kernel(x, edge_index, W1_self, W1_neigh, b1, W2_self, W2_neigh, b2):
    flat = lax.optimization_barrier(jnp.reshape(edge_index, (2 * N_EDGES,)))
    src_p = jnp.concatenate([flat[:N_EDGES], _PAD_SRC])
    dst_p = jnp.concatenate([flat[N_EDGES:], _PAD_DST])

    agg1, cnt = _SC_AGG_CNT(x, src_p, dst_p)
    b1r = b1.reshape(1, D)
    b2r = b2.reshape(1, D)

    h = _combine(agg1, cnt, x, W1_neigh, W1_self, b1r, relu=True)
    (agg2,) = _SC_AGG(h, src_p, dst_p)
    out = _combine(agg2, cnt, h, W2_neigh, W2_self, b2r, relu=False)
    return out
